# Initial kernel scaffold; baseline (speedup 1.0000x reference)
import jax
import jax.numpy as jnp
from jax.experimental import pallas as pl


def kernel(x, edge_index, W1_l, W1_r, b1, W2_l, W2_r, b2, Wfc, bfc):
    raise NotImplementedError("write your pallas kernel here")


# speedup vs baseline: 3.2955x; 3.2955x over previous
import functools

import jax
import jax.numpy as jnp
from jax import lax
from jax.experimental import pallas as pl
from jax.experimental.pallas import tpu as pltpu
from jax.experimental.pallas import tpu_sc as plsc

NUM_CORES = 2
NUM_SUBCORES = 16
NUM_WORKERS = NUM_CORES * NUM_SUBCORES
CHUNK = 128
DEG_W = 128
ZROWS = 64


def _ceil_to(x, m):
    return (x + m - 1) // m * m


def _l1_body(x_ref, w_ref, b_ref, p_ref, r_ref):
    acc = jnp.dot(x_ref[...], w_ref[...], preferred_element_type=jnp.float32)
    d = p_ref.shape[2]
    p_ref[0] = acc[:, :d]
    p_ref[1] = acc[:, d:2 * d]
    r_ref[...] = acc[:, 2 * d:] + b_ref[...]


def _l2_body(agg_ref, deg_ref, r1_ref, w_ref, b_ref, p2_ref, r2_ref):
    deg = jnp.maximum(deg_ref[0][:, :1] + deg_ref[1][:, :1], 1.0)
    agg = jnp.concatenate([agg_ref[0], agg_ref[1]], axis=1)
    h = jnp.maximum(agg / deg + r1_ref[...], 0.0)
    acc = jnp.dot(h, w_ref[...], preferred_element_type=jnp.float32)
    d = p2_ref.shape[1]
    p2_ref[...] = acc[:, :d]
    r2_ref[...] = acc[:, d:] + b_ref[...]


def _head_body(agg2_ref, deg_ref, r2_ref, w_ref, b_ref, o_ref):
    deg = jnp.maximum(deg_ref[0][:, :1] + deg_ref[1][:, :1], 1.0)
    h2 = (agg2_ref[0] + agg2_ref[1]) / deg + r2_ref[...]
    o = jnp.dot(h2, w_ref[...], preferred_element_type=jnp.float32) + b_ref[...]
    o_ref[...] = jnp.where(o >= 0, o, 0.01 * o)


def _row_spec(bm, cols):
    return pl.BlockSpec((bm, cols), lambda i: (i, 0))


def _stk_spec(bm, cols):
    return pl.BlockSpec((2, bm, cols), lambda i: (0, i, 0))


def _full_spec(shape):
    nd = len(shape)
    return pl.BlockSpec(shape, lambda i, nd=nd: (0,) * nd)


def _make_sc_deg(n_pad, n_chunks):
    rows_per_tile = n_pad // NUM_SUBCORES
    mesh = plsc.VectorSubcoreMesh(core_axis_name="c", subcore_axis_name="s")
    f32 = jnp.float32

    @functools.partial(
        pl.kernel,
        mesh=mesh,
        out_type=[
            jax.ShapeDtypeStruct((NUM_CORES, n_pad, DEG_W), f32),
        ],
        scratch_types=[
            pltpu.VMEM((n_chunks, CHUNK), jnp.int32),
            pltpu.VMEM((CHUNK, DEG_W), f32),
            pltpu.VMEM_SHARED((n_pad, DEG_W), f32),
        ],
    )
    def sc_kernel(dst_hbm, dzeros_hbm, ones_hbm, deg_hbm,
                  dst_v, ones_v, dacc_sh):
        c = lax.axis_index("c")
        s = lax.axis_index("s")
        wid = c * NUM_SUBCORES + s

        rsl = pl.ds(s * rows_per_tile, rows_per_tile)
        pltpu.sync_copy(dzeros_hbm, dacc_sh.at[rsl])
        pltpu.sync_copy(ones_hbm, ones_v)
        plsc.subcore_barrier()

        pltpu.sync_copy(dst_hbm.at[wid], dst_v)

        @pl.loop(0, n_chunks)
        def _(j):
            pltpu.sync_copy(ones_v, dacc_sh.at[dst_v.at[j]], add=True)

        plsc.subcore_barrier()
        pltpu.sync_copy(dacc_sh.at[rsl], deg_hbm.at[c].at[rsl])

    return sc_kernel


def _make_sc_layer1(n_pad, n_chunks, d_half):
    rows_per_tile = n_pad // NUM_SUBCORES
    mesh = plsc.VectorSubcoreMesh(core_axis_name="c", subcore_axis_name="s")
    f32 = jnp.float32

    @functools.partial(
        pl.kernel,
        mesh=mesh,
        out_type=[
            jax.ShapeDtypeStruct((NUM_CORES, n_pad, d_half), f32),
        ],
        scratch_types=[
            pltpu.VMEM((n_chunks, CHUNK), jnp.int32),
            pltpu.VMEM((n_chunks, CHUNK), jnp.int32),
            pltpu.VMEM((CHUNK, d_half), f32),
            pltpu.VMEM_SHARED((n_pad, d_half), f32),
            pltpu.SemaphoreType.DMA,
        ],
    )
    def sc_kernel(p_hbm, src_hbm, dst_hbm, zeros_hbm, agg_hbm,
                  src_v, dst_v, rows_v, acc_sh, sem):
        c = lax.axis_index("c")
        s = lax.axis_index("s")

        rsl = pl.ds(s * rows_per_tile, rows_per_tile)
        pltpu.sync_copy(zeros_hbm, acc_sh.at[rsl])

        plsc.subcore_barrier()

        p_c = p_hbm.at[c]
        for rr in range(2):
            row = s * 2 + rr
            pltpu.sync_copy(src_hbm.at[row], src_v)
            pltpu.sync_copy(dst_hbm.at[row], dst_v)

            @pl.loop(0, n_chunks)
            def _(j):
                pltpu.async_copy(p_c.at[src_v.at[j]], rows_v, sem).wait()
                pltpu.sync_copy(rows_v, acc_sh.at[dst_v.at[j]], add=True)

        plsc.subcore_barrier()

        pltpu.sync_copy(acc_sh.at[rsl], agg_hbm.at[c].at[rsl])

    return sc_kernel


def _make_sc_layer2(n_pad, n_chunks, d):
    rows_per_tile = n_pad // NUM_SUBCORES
    mesh = plsc.VectorSubcoreMesh(core_axis_name="c", subcore_axis_name="s")
    f32 = jnp.float32

    @functools.partial(
        pl.kernel,
        mesh=mesh,
        out_type=[
            jax.ShapeDtypeStruct((NUM_CORES, n_pad, d), f32),
        ],
        scratch_types=[
            pltpu.VMEM((n_chunks, CHUNK), jnp.int32),
            pltpu.VMEM((n_chunks, CHUNK), jnp.int32),
            pltpu.VMEM((CHUNK, d), f32),
            pltpu.VMEM_SHARED((n_pad, d), f32),
            pltpu.SemaphoreType.DMA,
        ],
    )
    def sc_kernel(p_hbm, src_hbm, dst_hbm, zeros_hbm, out_hbm,
                  src_v, dst_v, rows_v, acc_sh, sem):
        c = lax.axis_index("c")
        s = lax.axis_index("s")
        wid = c * NUM_SUBCORES + s

        rsl = pl.ds(s * rows_per_tile, rows_per_tile)
        pltpu.sync_copy(zeros_hbm, acc_sh.at[rsl])

        plsc.subcore_barrier()

        pltpu.sync_copy(src_hbm.at[wid], src_v)
        pltpu.sync_copy(dst_hbm.at[wid], dst_v)

        @pl.loop(0, n_chunks)
        def _(j):
            pltpu.async_copy(p_hbm.at[src_v.at[j]], rows_v, sem).wait()
            pltpu.sync_copy(rows_v, acc_sh.at[dst_v.at[j]], add=True)

        plsc.subcore_barrier()

        pltpu.sync_copy(acc_sh.at[rsl], out_hbm.at[c].at[rsl])

    return sc_kernel


def kernel(x, edge_index, W1_l, W1_r, b1, W2_l, W2_r, b2, Wfc, bfc):
    n, d_in = x.shape
    e = edge_index.shape[1]
    d_hid = W1_l.shape[1]
    d_out = W2_l.shape[1]
    f32 = jnp.float32

    n_pad = _ceil_to(n, NUM_SUBCORES * ZROWS)
    e_pad = _ceil_to(e, NUM_WORKERS * CHUNK)
    n_chunks = e_pad // (NUM_WORKERS * CHUNK)
    bm = 512
    grid_m = n_pad // bm
    d_half = d_hid // 2

    x_p = jnp.pad(x.astype(f32), ((0, n_pad - n), (0, 0)))
    src = edge_index[0].astype(jnp.int32)
    dst = edge_index[1].astype(jnp.int32)
    pad_e = e_pad - e
    src = jnp.concatenate([src, jnp.zeros((pad_e,), jnp.int32)])
    dst = jnp.concatenate([dst, jnp.full((pad_e,), n_pad - 1, jnp.int32)])
    src = src.reshape(NUM_WORKERS, n_chunks, CHUNK)
    dst = dst.reshape(NUM_WORKERS, n_chunks, CHUNK)

    w1 = jnp.concatenate([W1_l, W1_r], axis=1)
    b1_2d = b1.reshape(1, d_hid)
    w2 = jnp.concatenate([W2_l, W2_r], axis=1)
    b2_2d = b2.reshape(1, d_out)
    wfc_p = jnp.pad(Wfc, ((0, 0), (0, d_out - Wfc.shape[1])))
    bfc_p = jnp.pad(bfc, (0, d_out - bfc.shape[0])).reshape(1, d_out)

    p1, r1 = pl.pallas_call(
        _l1_body,
        grid=(grid_m,),
        in_specs=[_row_spec(bm, d_in), _full_spec(w1.shape),
                  _full_spec(b1_2d.shape)],
        out_specs=[_stk_spec(bm, d_half), _row_spec(bm, d_hid)],
        out_shape=[jax.ShapeDtypeStruct((NUM_CORES, n_pad, d_half), f32),
                   jax.ShapeDtypeStruct((n_pad, d_hid), f32)],
    )(x_p, w1, b1_2d)

    rows_per_tile = n_pad // NUM_SUBCORES
    zeros_h = jnp.zeros((rows_per_tile, d_half), f32)
    ones_h = jnp.ones((CHUNK, DEG_W), f32)

    scd = _make_sc_deg(n_pad, n_chunks)
    (deg,) = scd(dst, zeros_h, ones_h)

    sc1 = _make_sc_layer1(n_pad, n_chunks, d_half)
    (agg1,) = sc1(p1, src, dst, zeros_h)

    p2, r2 = pl.pallas_call(
        _l2_body,
        grid=(grid_m,),
        in_specs=[_stk_spec(bm, d_half), _stk_spec(bm, DEG_W),
                  _row_spec(bm, d_hid), _full_spec(w2.shape),
                  _full_spec(b2_2d.shape)],
        out_specs=[_row_spec(bm, d_out), _row_spec(bm, d_out)],
        out_shape=[jax.ShapeDtypeStruct((n_pad, d_out), f32),
                   jax.ShapeDtypeStruct((n_pad, d_out), f32)],
    )(agg1, deg, r1, w2, b2_2d)

    sc2 = _make_sc_layer2(n_pad, n_chunks, d_out)
    (agg2,) = sc2(p2, src, dst, zeros_h)

    out_p = pl.pallas_call(
        _head_body,
        grid=(grid_m,),
        in_specs=[_stk_spec(bm, d_out), _stk_spec(bm, DEG_W),
                  _row_spec(bm, d_out), _full_spec(wfc_p.shape),
                  _full_spec(bfc_p.shape)],
        out_specs=_row_spec(bm, d_out),
        out_shape=jax.ShapeDtypeStruct((n_pad, d_out), f32),
    )(agg2, deg, r2, wfc_p, bfc_p)

    return out_p[:n, :Wfc.shape[1]]


# speedup vs baseline: 3.5772x; 1.0855x over previous
import functools

import jax
import jax.numpy as jnp
from jax import lax
from jax.experimental import pallas as pl
from jax.experimental.pallas import tpu as pltpu
from jax.experimental.pallas import tpu_sc as plsc

NUM_CORES = 2
NUM_SUBCORES = 16
NUM_WORKERS = NUM_CORES * NUM_SUBCORES
CHUNK = 128
DEG_W = 128
ZROWS = 64


def _ceil_to(x, m):
    return (x + m - 1) // m * m


def _l1_body(x_ref, w_ref, b_ref, p_ref, r_ref):
    acc = jnp.dot(x_ref[...], w_ref[...], preferred_element_type=jnp.float32)
    d = p_ref.shape[2]
    p_ref[0] = acc[:, :d]
    p_ref[1] = acc[:, d:2 * d]
    r_ref[...] = acc[:, 2 * d:] + b_ref[...]


def _l2_body(agg_ref, deg_ref, r1_ref, w_ref, b_ref, p2_ref, r2_ref):
    deg = jnp.maximum(deg_ref[0][:, :1] + deg_ref[1][:, :1], 1.0)
    agg = jnp.concatenate([agg_ref[0], agg_ref[1]], axis=1)
    h = jnp.maximum(agg / deg + r1_ref[...], 0.0)
    acc = jnp.dot(h, w_ref[...], preferred_element_type=jnp.float32)
    d = p2_ref.shape[1]
    p2_ref[...] = acc[:, :d]
    r2_ref[...] = acc[:, d:] + b_ref[...]


def _head_body(agg2_ref, deg_ref, r2_ref, w_ref, b_ref, o_ref):
    deg = jnp.maximum(deg_ref[0][:, :1] + deg_ref[1][:, :1], 1.0)
    h2 = (agg2_ref[0] + agg2_ref[1]) / deg + r2_ref[...]
    o = jnp.dot(h2, w_ref[...], preferred_element_type=jnp.float32) + b_ref[...]
    o_ref[...] = jnp.where(o >= 0, o, 0.01 * o)


def _row_spec(bm, cols):
    return pl.BlockSpec((bm, cols), lambda i: (i, 0))


def _stk_spec(bm, cols):
    return pl.BlockSpec((2, bm, cols), lambda i: (0, i, 0))


def _full_spec(shape):
    nd = len(shape)
    return pl.BlockSpec(shape, lambda i, nd=nd: (0,) * nd)


def _edge_pass_pipelined(p_ref, src_v, dst_v, rows0, rows1, acc_sh,
                         gs0, gs1, ss0, ss1, n_chunks):
    def g(j, buf, sem):
        pltpu.async_copy(p_ref.at[src_v.at[j]], buf, sem)

    def gw(buf, sem):
        pltpu.make_async_copy(p_ref.at[src_v.at[0]], buf, sem).wait()

    def sct(j, buf, sem):
        pltpu.async_copy(buf, acc_sh.at[dst_v.at[j]], sem, add=True)

    def sw(buf, sem):
        pltpu.make_async_copy(buf, acc_sh.at[dst_v.at[0]], sem).wait()

    g(0, rows0, gs0)
    g(1, rows1, gs1)

    @pl.loop(0, n_chunks // 2 - 1)
    def _(t):
        j = 2 * t
        gw(rows0, gs0)
        sct(j, rows0, ss0)
        gw(rows1, gs1)
        sct(j + 1, rows1, ss1)
        sw(rows0, ss0)
        g(j + 2, rows0, gs0)
        sw(rows1, ss1)
        g(j + 3, rows1, gs1)

    gw(rows0, gs0)
    sct(n_chunks - 2, rows0, ss0)
    gw(rows1, gs1)
    sct(n_chunks - 1, rows1, ss1)
    sw(rows0, ss0)
    sw(rows1, ss1)


def _make_sc_deg(n_pad, n_chunks):
    rows_per_tile = n_pad // NUM_SUBCORES
    mesh = plsc.VectorSubcoreMesh(core_axis_name="c", subcore_axis_name="s")
    f32 = jnp.float32

    @functools.partial(
        pl.kernel,
        mesh=mesh,
        out_type=[
            jax.ShapeDtypeStruct((NUM_CORES, n_pad, DEG_W), f32),
        ],
        scratch_types=[
            pltpu.VMEM((n_chunks, CHUNK), jnp.int32),
            pltpu.VMEM((CHUNK, DEG_W), f32),
            pltpu.VMEM_SHARED((n_pad, DEG_W), f32),
            pltpu.SemaphoreType.DMA,
        ],
    )
    def sc_kernel(dst_hbm, dzeros_hbm, ones_hbm, deg_hbm,
                  dst_v, ones_v, dacc_sh, sem):
        c = lax.axis_index("c")
        s = lax.axis_index("s")
        wid = c * NUM_SUBCORES + s

        rsl = pl.ds(s * rows_per_tile, rows_per_tile)
        pltpu.sync_copy(dzeros_hbm, dacc_sh.at[rsl])
        pltpu.sync_copy(ones_hbm, ones_v)
        plsc.subcore_barrier()

        pltpu.sync_copy(dst_hbm.at[wid], dst_v)

        @pl.loop(0, n_chunks)
        def _(j):
            pltpu.async_copy(ones_v, dacc_sh.at[dst_v.at[j]], sem, add=True)

        @pl.loop(0, n_chunks)
        def _(j):
            pltpu.make_async_copy(ones_v, dacc_sh.at[dst_v.at[0]], sem).wait()

        plsc.subcore_barrier()
        pltpu.sync_copy(dacc_sh.at[rsl], deg_hbm.at[c].at[rsl])

    return sc_kernel


def _make_sc_layer1(n_pad, n_chunks, d_half):
    rows_per_tile = n_pad // NUM_SUBCORES
    mesh = plsc.VectorSubcoreMesh(core_axis_name="c", subcore_axis_name="s")
    f32 = jnp.float32

    @functools.partial(
        pl.kernel,
        mesh=mesh,
        out_type=[
            jax.ShapeDtypeStruct((NUM_CORES, n_pad, d_half), f32),
        ],
        scratch_types=[
            pltpu.VMEM((n_chunks, CHUNK), jnp.int32),
            pltpu.VMEM((n_chunks, CHUNK), jnp.int32),
            pltpu.VMEM((CHUNK, d_half), f32),
            pltpu.VMEM((CHUNK, d_half), f32),
            pltpu.VMEM_SHARED((n_pad, d_half), f32),
            pltpu.SemaphoreType.DMA,
            pltpu.SemaphoreType.DMA,
            pltpu.SemaphoreType.DMA,
            pltpu.SemaphoreType.DMA,
        ],
    )
    def sc_kernel(p_hbm, src_hbm, dst_hbm, zeros_hbm, agg_hbm,
                  src_v, dst_v, rows0, rows1, acc_sh, gs0, gs1, ss0, ss1):
        c = lax.axis_index("c")
        s = lax.axis_index("s")

        rsl = pl.ds(s * rows_per_tile, rows_per_tile)
        pltpu.sync_copy(zeros_hbm, acc_sh.at[rsl])

        plsc.subcore_barrier()

        p_c = p_hbm.at[c]
        for rr in range(2):
            row = s * 2 + rr
            pltpu.sync_copy(src_hbm.at[row], src_v)
            pltpu.sync_copy(dst_hbm.at[row], dst_v)
            _edge_pass_pipelined(p_c, src_v, dst_v, rows0, rows1, acc_sh,
                                 gs0, gs1, ss0, ss1, n_chunks)

        plsc.subcore_barrier()

        pltpu.sync_copy(acc_sh.at[rsl], agg_hbm.at[c].at[rsl])

    return sc_kernel


def _make_sc_layer2(n_pad, n_chunks, d):
    rows_per_tile = n_pad // NUM_SUBCORES
    mesh = plsc.VectorSubcoreMesh(core_axis_name="c", subcore_axis_name="s")
    f32 = jnp.float32

    @functools.partial(
        pl.kernel,
        mesh=mesh,
        out_type=[
            jax.ShapeDtypeStruct((NUM_CORES, n_pad, d), f32),
        ],
        scratch_types=[
            pltpu.VMEM((n_chunks, CHUNK), jnp.int32),
            pltpu.VMEM((n_chunks, CHUNK), jnp.int32),
            pltpu.VMEM((CHUNK, d), f32),
            pltpu.VMEM((CHUNK, d), f32),
            pltpu.VMEM_SHARED((n_pad, d), f32),
            pltpu.SemaphoreType.DMA,
            pltpu.SemaphoreType.DMA,
            pltpu.SemaphoreType.DMA,
            pltpu.SemaphoreType.DMA,
        ],
    )
    def sc_kernel(p_hbm, src_hbm, dst_hbm, zeros_hbm, out_hbm,
                  src_v, dst_v, rows0, rows1, acc_sh, gs0, gs1, ss0, ss1):
        c = lax.axis_index("c")
        s = lax.axis_index("s")
        wid = c * NUM_SUBCORES + s

        rsl = pl.ds(s * rows_per_tile, rows_per_tile)
        pltpu.sync_copy(zeros_hbm, acc_sh.at[rsl])

        plsc.subcore_barrier()

        pltpu.sync_copy(src_hbm.at[wid], src_v)
        pltpu.sync_copy(dst_hbm.at[wid], dst_v)
        _edge_pass_pipelined(p_hbm, src_v, dst_v, rows0, rows1, acc_sh,
                             gs0, gs1, ss0, ss1, n_chunks)

        plsc.subcore_barrier()

        pltpu.sync_copy(acc_sh.at[rsl], out_hbm.at[c].at[rsl])

    return sc_kernel


def kernel(x, edge_index, W1_l, W1_r, b1, W2_l, W2_r, b2, Wfc, bfc):
    n, d_in = x.shape
    e = edge_index.shape[1]
    d_hid = W1_l.shape[1]
    d_out = W2_l.shape[1]
    f32 = jnp.float32

    n_pad = _ceil_to(n, NUM_SUBCORES * ZROWS)
    e_pad = _ceil_to(e, NUM_WORKERS * CHUNK)
    n_chunks = e_pad // (NUM_WORKERS * CHUNK)
    bm = 512
    grid_m = n_pad // bm
    d_half = d_hid // 2

    x_p = jnp.pad(x.astype(f32), ((0, n_pad - n), (0, 0)))
    src = edge_index[0].astype(jnp.int32)
    dst = edge_index[1].astype(jnp.int32)
    pad_e = e_pad - e
    src = jnp.concatenate([src, jnp.zeros((pad_e,), jnp.int32)])
    dst = jnp.concatenate([dst, jnp.full((pad_e,), n_pad - 1, jnp.int32)])
    src = src.reshape(NUM_WORKERS, n_chunks, CHUNK)
    dst = dst.reshape(NUM_WORKERS, n_chunks, CHUNK)

    w1 = jnp.concatenate([W1_l, W1_r], axis=1)
    b1_2d = b1.reshape(1, d_hid)
    w2 = jnp.concatenate([W2_l, W2_r], axis=1)
    b2_2d = b2.reshape(1, d_out)
    wfc_p = jnp.pad(Wfc, ((0, 0), (0, d_out - Wfc.shape[1])))
    bfc_p = jnp.pad(bfc, (0, d_out - bfc.shape[0])).reshape(1, d_out)

    p1, r1 = pl.pallas_call(
        _l1_body,
        grid=(grid_m,),
        in_specs=[_row_spec(bm, d_in), _full_spec(w1.shape),
                  _full_spec(b1_2d.shape)],
        out_specs=[_stk_spec(bm, d_half), _row_spec(bm, d_hid)],
        out_shape=[jax.ShapeDtypeStruct((NUM_CORES, n_pad, d_half), f32),
                   jax.ShapeDtypeStruct((n_pad, d_hid), f32)],
    )(x_p, w1, b1_2d)

    rows_per_tile = n_pad // NUM_SUBCORES
    zeros_h = jnp.zeros((rows_per_tile, d_half), f32)
    ones_h = jnp.ones((CHUNK, DEG_W), f32)

    scd = _make_sc_deg(n_pad, n_chunks)
    (deg,) = scd(dst, zeros_h, ones_h)

    sc1 = _make_sc_layer1(n_pad, n_chunks, d_half)
    (agg1,) = sc1(p1, src, dst, zeros_h)

    p2, r2 = pl.pallas_call(
        _l2_body,
        grid=(grid_m,),
        in_specs=[_stk_spec(bm, d_half), _stk_spec(bm, DEG_W),
                  _row_spec(bm, d_hid), _full_spec(w2.shape),
                  _full_spec(b2_2d.shape)],
        out_specs=[_row_spec(bm, d_out), _row_spec(bm, d_out)],
        out_shape=[jax.ShapeDtypeStruct((n_pad, d_out), f32),
                   jax.ShapeDtypeStruct((n_pad, d_out), f32)],
    )(agg1, deg, r1, w2, b2_2d)

    sc2 = _make_sc_layer2(n_pad, n_chunks, d_out)
    (agg2,) = sc2(p2, src, dst, zeros_h)

    out_p = pl.pallas_call(
        _head_body,
        grid=(grid_m,),
        in_specs=[_stk_spec(bm, d_out), _stk_spec(bm, DEG_W),
                  _row_spec(bm, d_out), _full_spec(wfc_p.shape),
                  _full_spec(bfc_p.shape)],
        out_specs=_row_spec(bm, d_out),
        out_shape=jax.ShapeDtypeStruct((n_pad, d_out), f32),
    )(agg2, deg, r2, wfc_p, bfc_p)

    return out_p[:n, :Wfc.shape[1]]


# speedup vs baseline: 6.9469x; 1.9420x over previous
import functools

import jax
import jax.numpy as jnp
from jax import lax
from jax.experimental import pallas as pl
from jax.experimental.pallas import tpu as pltpu
from jax.experimental.pallas import tpu_sc as plsc

NUM_CORES = 2
NUM_SUBCORES = 16
NUM_WORKERS = NUM_CORES * NUM_SUBCORES
CHUNK = 128
DEG_W = 128
ZROWS = 64


def _ceil_to(x, m):
    return (x + m - 1) // m * m


def _l1_body(x_ref, w_ref, b_ref, p_ref, r_ref):
    acc = jnp.dot(x_ref[...], w_ref[...], preferred_element_type=jnp.float32)
    d = p_ref.shape[2]
    p_ref[0] = acc[:, :d]
    p_ref[1] = acc[:, d:2 * d]
    r_ref[...] = acc[:, 2 * d:] + b_ref[...]


def _l2_body(agg_ref, deg_ref, r1_ref, w_ref, b_ref, p2_ref, r2_ref):
    deg = jnp.maximum(deg_ref[0][:, :1] + deg_ref[1][:, :1], 1.0)
    agg = jnp.concatenate([agg_ref[0], agg_ref[1]], axis=1)
    h = jnp.maximum(agg / deg + r1_ref[...], 0.0)
    acc = jnp.dot(h, w_ref[...], preferred_element_type=jnp.float32)
    d = p2_ref.shape[1]
    p2_ref[...] = acc[:, :d]
    r2_ref[...] = acc[:, d:] + b_ref[...]


def _head_body(agg2_ref, deg_ref, r2_ref, w_ref, b_ref, o_ref):
    deg = jnp.maximum(deg_ref[0][:, :1] + deg_ref[1][:, :1], 1.0)
    h2 = (agg2_ref[0] + agg2_ref[1]) / deg + r2_ref[...]
    o = jnp.dot(h2, w_ref[...], preferred_element_type=jnp.float32) + b_ref[...]
    o_ref[...] = jnp.where(o >= 0, o, 0.01 * o)


def _row_spec(bm, cols):
    return pl.BlockSpec((bm, cols), lambda i: (i, 0))


def _stk_spec(bm, cols):
    return pl.BlockSpec((2, bm, cols), lambda i: (0, i, 0))


def _full_spec(shape):
    nd = len(shape)
    return pl.BlockSpec(shape, lambda i, nd=nd: (0,) * nd)


def _edge_pass_pipelined(p_ref, src_v, dst_v, rows0, rows1, acc_sh,
                         gs0, gs1, ss0, ss1, n_chunks):
    def g(j, buf, sem):
        pltpu.async_copy(p_ref.at[src_v.at[j]], buf, sem)

    def gw(buf, sem):
        pltpu.make_async_copy(p_ref.at[src_v.at[0]], buf, sem).wait()

    def sct(j, buf, sem):
        pltpu.async_copy(buf, acc_sh.at[dst_v.at[j]], sem, add=True)

    def sw(buf, sem):
        pltpu.make_async_copy(buf, acc_sh.at[dst_v.at[0]], sem).wait()

    g(0, rows0, gs0)
    g(1, rows1, gs1)

    @pl.loop(0, n_chunks // 2 - 1)
    def _(t):
        j = 2 * t
        gw(rows0, gs0)
        sct(j, rows0, ss0)
        gw(rows1, gs1)
        sct(j + 1, rows1, ss1)
        sw(rows0, ss0)
        g(j + 2, rows0, gs0)
        sw(rows1, ss1)
        g(j + 3, rows1, gs1)

    gw(rows0, gs0)
    sct(n_chunks - 2, rows0, ss0)
    gw(rows1, gs1)
    sct(n_chunks - 1, rows1, ss1)
    sw(rows0, ss0)
    sw(rows1, ss1)


def _make_sc_deg(n_pad, n_chunks):
    rows_per_tile = n_pad // NUM_SUBCORES
    mesh = plsc.VectorSubcoreMesh(core_axis_name="c", subcore_axis_name="s")
    f32 = jnp.float32

    @functools.partial(
        pl.kernel,
        mesh=mesh,
        out_type=[
            jax.ShapeDtypeStruct((NUM_CORES, n_pad, DEG_W), f32),
        ],
        scratch_types=[
            pltpu.VMEM((n_chunks, CHUNK), jnp.int32),
            pltpu.VMEM((CHUNK, DEG_W), f32),
            pltpu.VMEM_SHARED((n_pad, DEG_W), f32),
            pltpu.SemaphoreType.DMA,
        ],
    )
    def sc_kernel(dst_hbm, dzeros_hbm, ones_hbm, deg_hbm,
                  dst_v, ones_v, dacc_sh, sem):
        c = lax.axis_index("c")
        s = lax.axis_index("s")
        wid = c * NUM_SUBCORES + s

        rsl = pl.ds(s * rows_per_tile, rows_per_tile)
        pltpu.sync_copy(dzeros_hbm, dacc_sh.at[rsl])
        pltpu.sync_copy(ones_hbm, ones_v)
        plsc.subcore_barrier()

        pltpu.sync_copy(dst_hbm.at[wid], dst_v)

        @pl.loop(0, n_chunks)
        def _(j):
            pltpu.async_copy(ones_v, dacc_sh.at[dst_v.at[j]], sem, add=True)

        @pl.loop(0, n_chunks)
        def _(j):
            pltpu.make_async_copy(ones_v, dacc_sh.at[dst_v.at[0]], sem).wait()

        plsc.subcore_barrier()
        pltpu.sync_copy(dacc_sh.at[rsl], deg_hbm.at[c].at[rsl])

    return sc_kernel


def _make_sc_layer1(n_pad, n_chunks, d_half):
    rows_per_tile = n_pad // NUM_SUBCORES
    mesh = plsc.VectorSubcoreMesh(core_axis_name="c", subcore_axis_name="s")
    f32 = jnp.float32

    @functools.partial(
        pl.kernel,
        mesh=mesh,
        out_type=[
            jax.ShapeDtypeStruct((NUM_CORES, n_pad, d_half), f32),
        ],
        scratch_types=[
            pltpu.VMEM((n_chunks, CHUNK), jnp.int32),
            pltpu.VMEM((n_chunks, CHUNK), jnp.int32),
            pltpu.VMEM((CHUNK, d_half), f32),
            pltpu.VMEM((CHUNK, d_half), f32),
            pltpu.VMEM_SHARED((n_pad, d_half), f32),
            pltpu.SemaphoreType.DMA,
            pltpu.SemaphoreType.DMA,
            pltpu.SemaphoreType.DMA,
            pltpu.SemaphoreType.DMA,
        ],
    )
    def sc_kernel(p_hbm, src_hbm, dst_hbm, zeros_hbm, agg_hbm,
                  src_v, dst_v, rows0, rows1, acc_sh, gs0, gs1, ss0, ss1):
        c = lax.axis_index("c")
        s = lax.axis_index("s")

        rsl = pl.ds(s * rows_per_tile, rows_per_tile)
        pltpu.sync_copy(zeros_hbm, acc_sh.at[rsl])

        plsc.subcore_barrier()

        p_c = p_hbm.at[c]
        for rr in range(2):
            row = s * 2 + rr
            pltpu.sync_copy(src_hbm.at[row], src_v)
            pltpu.sync_copy(dst_hbm.at[row], dst_v)
            _edge_pass_pipelined(p_c, src_v, dst_v, rows0, rows1, acc_sh,
                                 gs0, gs1, ss0, ss1, n_chunks)

        plsc.subcore_barrier()

        pltpu.sync_copy(acc_sh.at[rsl], agg_hbm.at[c].at[rsl])

    return sc_kernel


def _make_sc_layer2(n_pad, n_chunks, d):
    rows_per_tile = n_pad // NUM_SUBCORES
    mesh = plsc.VectorSubcoreMesh(core_axis_name="c", subcore_axis_name="s")
    f32 = jnp.float32

    @functools.partial(
        pl.kernel,
        mesh=mesh,
        out_type=[
            jax.ShapeDtypeStruct((NUM_CORES, n_pad, d), f32),
        ],
        scratch_types=[
            pltpu.VMEM((n_chunks, CHUNK), jnp.int32),
            pltpu.VMEM((n_chunks, CHUNK), jnp.int32),
            pltpu.VMEM((CHUNK, d), f32),
            pltpu.VMEM((CHUNK, d), f32),
            pltpu.VMEM_SHARED((n_pad, d), f32),
            pltpu.SemaphoreType.DMA,
            pltpu.SemaphoreType.DMA,
            pltpu.SemaphoreType.DMA,
            pltpu.SemaphoreType.DMA,
        ],
    )
    def sc_kernel(p_hbm, src_hbm, dst_hbm, zeros_hbm, out_hbm,
                  src_v, dst_v, rows0, rows1, acc_sh, gs0, gs1, ss0, ss1):
        c = lax.axis_index("c")
        s = lax.axis_index("s")
        wid = c * NUM_SUBCORES + s

        rsl = pl.ds(s * rows_per_tile, rows_per_tile)
        pltpu.sync_copy(zeros_hbm, acc_sh.at[rsl])

        plsc.subcore_barrier()

        pltpu.sync_copy(src_hbm.at[wid], src_v)
        pltpu.sync_copy(dst_hbm.at[wid], dst_v)
        _edge_pass_pipelined(p_hbm, src_v, dst_v, rows0, rows1, acc_sh,
                             gs0, gs1, ss0, ss1, n_chunks)

        plsc.subcore_barrier()

        pltpu.sync_copy(acc_sh.at[rsl], out_hbm.at[c].at[rsl])

    return sc_kernel


def kernel(x, edge_index, W1_l, W1_r, b1, W2_l, W2_r, b2, Wfc, bfc):
    n, d_in = x.shape
    e = edge_index.shape[1]
    d_hid = W1_l.shape[1]
    d_out = W2_l.shape[1]
    f32 = jnp.float32

    n_pad = _ceil_to(n + 1, NUM_SUBCORES * ZROWS)
    e_pad = _ceil_to(e, NUM_WORKERS * CHUNK)
    n_chunks = e_pad // (NUM_WORKERS * CHUNK)
    bm = 512
    grid_m = n_pad // bm
    d_half = d_hid // 2

    x_p = jnp.pad(x.astype(f32), ((0, n_pad - n), (0, 0)))
    src = edge_index[0].astype(jnp.int32)
    dst = edge_index[1].astype(jnp.int32)
    pad_e = e_pad - e
    n_scratch = max(n_pad - n, 1)
    pad_i = jnp.arange(pad_e, dtype=jnp.int32)
    src = jnp.concatenate([src, pad_i % jnp.int32(n)])
    dst = jnp.concatenate([dst, (n_pad - n_scratch) + pad_i % jnp.int32(n_scratch)])
    src = src.reshape(NUM_WORKERS, n_chunks, CHUNK)
    dst = dst.reshape(NUM_WORKERS, n_chunks, CHUNK)

    w1 = jnp.concatenate([W1_l, W1_r], axis=1)
    b1_2d = b1.reshape(1, d_hid)
    w2 = jnp.concatenate([W2_l, W2_r], axis=1)
    b2_2d = b2.reshape(1, d_out)
    wfc_p = jnp.pad(Wfc, ((0, 0), (0, d_out - Wfc.shape[1])))
    bfc_p = jnp.pad(bfc, (0, d_out - bfc.shape[0])).reshape(1, d_out)

    p1, r1 = pl.pallas_call(
        _l1_body,
        grid=(grid_m,),
        in_specs=[_row_spec(bm, d_in), _full_spec(w1.shape),
                  _full_spec(b1_2d.shape)],
        out_specs=[_stk_spec(bm, d_half), _row_spec(bm, d_hid)],
        out_shape=[jax.ShapeDtypeStruct((NUM_CORES, n_pad, d_half), f32),
                   jax.ShapeDtypeStruct((n_pad, d_hid), f32)],
    )(x_p, w1, b1_2d)

    rows_per_tile = n_pad // NUM_SUBCORES
    zeros_h = jnp.zeros((rows_per_tile, d_half), f32)
    ones_h = jnp.ones((CHUNK, DEG_W), f32)

    scd = _make_sc_deg(n_pad, n_chunks)
    (deg,) = scd(dst, zeros_h, ones_h)

    sc1 = _make_sc_layer1(n_pad, n_chunks, d_half)
    (agg1,) = sc1(p1, src, dst, zeros_h)

    p2, r2 = pl.pallas_call(
        _l2_body,
        grid=(grid_m,),
        in_specs=[_stk_spec(bm, d_half), _stk_spec(bm, DEG_W),
                  _row_spec(bm, d_hid), _full_spec(w2.shape),
                  _full_spec(b2_2d.shape)],
        out_specs=[_row_spec(bm, d_out), _row_spec(bm, d_out)],
        out_shape=[jax.ShapeDtypeStruct((n_pad, d_out), f32),
                   jax.ShapeDtypeStruct((n_pad, d_out), f32)],
    )(agg1, deg, r1, w2, b2_2d)

    sc2 = _make_sc_layer2(n_pad, n_chunks, d_out)
    (agg2,) = sc2(p2, src, dst, zeros_h)

    out_p = pl.pallas_call(
        _head_body,
        grid=(grid_m,),
        in_specs=[_stk_spec(bm, d_out), _stk_spec(bm, DEG_W),
                  _row_spec(bm, d_out), _full_spec(wfc_p.shape),
                  _full_spec(bfc_p.shape)],
        out_specs=_row_spec(bm, d_out),
        out_shape=jax.ShapeDtypeStruct((n_pad, d_out), f32),
    )(agg2, deg, r2, wfc_p, bfc_p)

    return out_p[:n, :Wfc.shape[1]]


# speedup vs baseline: 6.9667x; 1.0028x over previous
import functools

import jax
import jax.numpy as jnp
from jax import lax
from jax.experimental import pallas as pl
from jax.experimental.pallas import tpu as pltpu
from jax.experimental.pallas import tpu_sc as plsc

NUM_CORES = 2
NUM_SUBCORES = 16
NUM_WORKERS = NUM_CORES * NUM_SUBCORES
CHUNK = 128
DEG_W = 128
ZROWS = 64


def _ceil_to(x, m):
    return (x + m - 1) // m * m


def _l1_body(x_ref, w_ref, b_ref, p_ref, r_ref):
    acc = jnp.dot(x_ref[...], w_ref[...], preferred_element_type=jnp.float32)
    d = p_ref.shape[2]
    p_ref[0] = acc[:, :d]
    p_ref[1] = acc[:, d:2 * d]
    r_ref[...] = acc[:, 2 * d:] + b_ref[...]


def _l2_body(agg_ref, deg_ref, r1_ref, w_ref, b_ref, p2_ref, r2_ref):
    deg = jnp.maximum(deg_ref[0][:, :1] + deg_ref[1][:, :1], 1.0)
    agg = jnp.concatenate([agg_ref[0], agg_ref[1]], axis=1)
    h = jnp.maximum(agg / deg + r1_ref[...], 0.0)
    acc = jnp.dot(h, w_ref[...], preferred_element_type=jnp.float32)
    d = p2_ref.shape[1]
    p2_ref[...] = acc[:, :d]
    r2_ref[...] = acc[:, d:] + b_ref[...]


def _head_body(agg2_ref, deg_ref, r2_ref, w_ref, b_ref, o_ref):
    deg = jnp.maximum(deg_ref[0][:, :1] + deg_ref[1][:, :1], 1.0)
    h2 = (agg2_ref[0] + agg2_ref[1]) / deg + r2_ref[...]
    o = jnp.dot(h2, w_ref[...], preferred_element_type=jnp.float32) + b_ref[...]
    o_ref[...] = jnp.where(o >= 0, o, 0.01 * o)


def _row_spec(bm, cols):
    return pl.BlockSpec((bm, cols), lambda i: (i, 0))


def _stk_spec(bm, cols):
    return pl.BlockSpec((2, bm, cols), lambda i: (0, i, 0))


def _full_spec(shape):
    nd = len(shape)
    return pl.BlockSpec(shape, lambda i, nd=nd: (0,) * nd)


def _edge_pass_pipelined(p_ref, src_v, dst_v, rows0, rows1, acc_sh,
                         gs0, gs1, ss0, ss1, n_chunks):
    def g(j, buf, sem):
        pltpu.async_copy(p_ref.at[src_v.at[j]], buf, sem)

    def gw(buf, sem):
        pltpu.make_async_copy(p_ref.at[src_v.at[0]], buf, sem).wait()

    def sct(j, buf, sem):
        pltpu.async_copy(buf, acc_sh.at[dst_v.at[j]], sem, add=True)

    def sw(buf, sem):
        pltpu.make_async_copy(buf, acc_sh.at[dst_v.at[0]], sem).wait()

    g(0, rows0, gs0)
    g(1, rows1, gs1)

    @pl.loop(0, n_chunks // 2 - 1)
    def _(t):
        j = 2 * t
        gw(rows0, gs0)
        sct(j, rows0, ss0)
        gw(rows1, gs1)
        sct(j + 1, rows1, ss1)
        sw(rows0, ss0)
        g(j + 2, rows0, gs0)
        sw(rows1, ss1)
        g(j + 3, rows1, gs1)

    gw(rows0, gs0)
    sct(n_chunks - 2, rows0, ss0)
    gw(rows1, gs1)
    sct(n_chunks - 1, rows1, ss1)
    sw(rows0, ss0)
    sw(rows1, ss1)


def _make_sc_deg(n_pad, n_chunks):
    rows_per_tile = n_pad // NUM_SUBCORES
    mesh = plsc.VectorSubcoreMesh(core_axis_name="c", subcore_axis_name="s")
    f32 = jnp.float32

    @functools.partial(
        pl.kernel,
        mesh=mesh,
        out_type=[
            jax.ShapeDtypeStruct((NUM_CORES, n_pad, DEG_W), f32),
        ],
        scratch_types=[
            pltpu.VMEM((n_chunks, CHUNK), jnp.int32),
            pltpu.VMEM((CHUNK, DEG_W), f32),
            pltpu.VMEM_SHARED((n_pad, DEG_W), f32),
            pltpu.SemaphoreType.DMA,
        ],
    )
    def sc_kernel(dst_hbm, dzeros_hbm, ones_hbm, deg_hbm,
                  dst_v, ones_v, dacc_sh, sem):
        c = lax.axis_index("c")
        s = lax.axis_index("s")
        wid = c * NUM_SUBCORES + s

        rsl = pl.ds(s * rows_per_tile, rows_per_tile)
        pltpu.sync_copy(dzeros_hbm, dacc_sh.at[rsl])
        pltpu.sync_copy(ones_hbm, ones_v)
        plsc.subcore_barrier()

        pltpu.sync_copy(dst_hbm.at[wid], dst_v)

        @pl.loop(0, n_chunks)
        def _(j):
            pltpu.async_copy(ones_v, dacc_sh.at[dst_v.at[j]], sem, add=True)

        @pl.loop(0, n_chunks)
        def _(j):
            pltpu.make_async_copy(ones_v, dacc_sh.at[dst_v.at[0]], sem).wait()

        plsc.subcore_barrier()
        pltpu.sync_copy(dacc_sh.at[rsl], deg_hbm.at[c].at[rsl])

    return sc_kernel


def _make_sc_layer1(n_pad, n_chunks, d_half):
    rows_per_tile = n_pad // NUM_SUBCORES
    mesh = plsc.VectorSubcoreMesh(core_axis_name="c", subcore_axis_name="s")
    f32 = jnp.float32

    @functools.partial(
        pl.kernel,
        mesh=mesh,
        out_type=[
            jax.ShapeDtypeStruct((NUM_CORES, n_pad, d_half), f32),
        ],
        scratch_types=[
            pltpu.VMEM((n_chunks, CHUNK), jnp.int32),
            pltpu.VMEM((n_chunks, CHUNK), jnp.int32),
            pltpu.VMEM((CHUNK, d_half), f32),
            pltpu.VMEM((CHUNK, d_half), f32),
            pltpu.VMEM_SHARED((n_pad, d_half), f32),
            pltpu.SemaphoreType.DMA,
            pltpu.SemaphoreType.DMA,
            pltpu.SemaphoreType.DMA,
            pltpu.SemaphoreType.DMA,
        ],
    )
    def sc_kernel(p_hbm, src_hbm, dst_hbm, zeros_hbm, agg_hbm,
                  src_v, dst_v, rows0, rows1, acc_sh, gs0, gs1, ss0, ss1):
        c = lax.axis_index("c")
        s = lax.axis_index("s")

        rsl = pl.ds(s * rows_per_tile, rows_per_tile)
        pltpu.sync_copy(zeros_hbm, acc_sh.at[rsl])

        plsc.subcore_barrier()

        p_c = p_hbm.at[c]
        for rr in range(2):
            row = s * 2 + rr
            pltpu.sync_copy(src_hbm.at[row], src_v)
            pltpu.sync_copy(dst_hbm.at[row], dst_v)
            _edge_pass_pipelined(p_c, src_v, dst_v, rows0, rows1, acc_sh,
                                 gs0, gs1, ss0, ss1, n_chunks)

        plsc.subcore_barrier()

        pltpu.sync_copy(acc_sh.at[rsl], agg_hbm.at[c].at[rsl])

    return sc_kernel


def _make_sc_layer2(n_pad, n_chunks, d):
    rows_per_tile = n_pad // NUM_SUBCORES
    mesh = plsc.VectorSubcoreMesh(core_axis_name="c", subcore_axis_name="s")
    f32 = jnp.float32

    @functools.partial(
        pl.kernel,
        mesh=mesh,
        out_type=[
            jax.ShapeDtypeStruct((NUM_CORES, n_pad, d), f32),
        ],
        scratch_types=[
            pltpu.VMEM((n_chunks, CHUNK), jnp.int32),
            pltpu.VMEM((n_chunks, CHUNK), jnp.int32),
            pltpu.VMEM((CHUNK, d), f32),
            pltpu.VMEM((CHUNK, d), f32),
            pltpu.VMEM_SHARED((n_pad, d), f32),
            pltpu.SemaphoreType.DMA,
            pltpu.SemaphoreType.DMA,
            pltpu.SemaphoreType.DMA,
            pltpu.SemaphoreType.DMA,
        ],
    )
    def sc_kernel(p_hbm, src_hbm, dst_hbm, zeros_hbm, out_hbm,
                  src_v, dst_v, rows0, rows1, acc_sh, gs0, gs1, ss0, ss1):
        c = lax.axis_index("c")
        s = lax.axis_index("s")
        wid = c * NUM_SUBCORES + s

        rsl = pl.ds(s * rows_per_tile, rows_per_tile)
        pltpu.sync_copy(zeros_hbm, acc_sh.at[rsl])

        plsc.subcore_barrier()

        pltpu.sync_copy(src_hbm.at[wid], src_v)
        pltpu.sync_copy(dst_hbm.at[wid], dst_v)
        _edge_pass_pipelined(p_hbm, src_v, dst_v, rows0, rows1, acc_sh,
                             gs0, gs1, ss0, ss1, n_chunks)

        plsc.subcore_barrier()

        pltpu.sync_copy(acc_sh.at[rsl], out_hbm.at[c].at[rsl])

    return sc_kernel


def kernel(x, edge_index, W1_l, W1_r, b1, W2_l, W2_r, b2, Wfc, bfc):
    n, d_in = x.shape
    e = edge_index.shape[1]
    d_hid = W1_l.shape[1]
    d_out = W2_l.shape[1]
    f32 = jnp.float32

    n_pad = _ceil_to(n + 1, NUM_SUBCORES * ZROWS)
    e_pad = _ceil_to(e, NUM_WORKERS * CHUNK)
    n_chunks = e_pad // (NUM_WORKERS * CHUNK)
    bm = 512
    grid_m = n_pad // bm
    d_half = d_hid // 2

    x_p = jnp.pad(x.astype(f32), ((0, n_pad - n), (0, 0)))
    src = edge_index[0].astype(jnp.int32)
    dst = edge_index[1].astype(jnp.int32)
    pad_e = e_pad - e
    n_scratch = max(n_pad - n, 1)
    pad_i = jnp.arange(pad_e, dtype=jnp.int32)
    src = jnp.concatenate([src, pad_i % jnp.int32(n)])
    dst = jnp.concatenate([dst, (n_pad - n_scratch) + pad_i % jnp.int32(n_scratch)])
    src = src.reshape(NUM_WORKERS, n_chunks, CHUNK)
    dst = dst.reshape(NUM_WORKERS, n_chunks, CHUNK)

    w1 = jnp.concatenate([W1_l, W1_r], axis=1)
    b1_2d = b1.reshape(1, d_hid)
    w2 = jnp.concatenate([W2_l, W2_r], axis=1)
    b2_2d = b2.reshape(1, d_out)
    wfc_p = jnp.pad(Wfc, ((0, 0), (0, d_out - Wfc.shape[1])))
    bfc_p = jnp.pad(bfc, (0, d_out - bfc.shape[0])).reshape(1, d_out)

    rows_per_tile = n_pad // NUM_SUBCORES
    zeros_h = jnp.zeros((rows_per_tile, d_half), f32)
    ones_h = jnp.ones((CHUNK, DEG_W), f32)

    scd = _make_sc_deg(n_pad, n_chunks)
    (deg,) = scd(dst, zeros_h, ones_h)

    p1, r1 = pl.pallas_call(
        _l1_body,
        grid=(grid_m,),
        in_specs=[_row_spec(bm, d_in), _full_spec(w1.shape),
                  _full_spec(b1_2d.shape)],
        out_specs=[_stk_spec(bm, d_half), _row_spec(bm, d_hid)],
        out_shape=[jax.ShapeDtypeStruct((NUM_CORES, n_pad, d_half), f32),
                   jax.ShapeDtypeStruct((n_pad, d_hid), f32)],
    )(x_p, w1, b1_2d)

    sc1 = _make_sc_layer1(n_pad, n_chunks, d_half)
    (agg1,) = sc1(p1, src, dst, zeros_h)

    p2, r2 = pl.pallas_call(
        _l2_body,
        grid=(grid_m,),
        in_specs=[_stk_spec(bm, d_half), _stk_spec(bm, DEG_W),
                  _row_spec(bm, d_hid), _full_spec(w2.shape),
                  _full_spec(b2_2d.shape)],
        out_specs=[_row_spec(bm, d_out), _row_spec(bm, d_out)],
        out_shape=[jax.ShapeDtypeStruct((n_pad, d_out), f32),
                   jax.ShapeDtypeStruct((n_pad, d_out), f32)],
    )(agg1, deg, r1, w2, b2_2d)

    sc2 = _make_sc_layer2(n_pad, n_chunks, d_out)
    (agg2,) = sc2(p2, src, dst, zeros_h)

    out_p = pl.pallas_call(
        _head_body,
        grid=(grid_m,),
        in_specs=[_stk_spec(bm, d_out), _stk_spec(bm, DEG_W),
                  _row_spec(bm, d_out), _full_spec(wfc_p.shape),
                  _full_spec(bfc_p.shape)],
        out_specs=_row_spec(bm, d_out),
        out_shape=jax.ShapeDtypeStruct((n_pad, d_out), f32),
    )(agg2, deg, r2, wfc_p, bfc_p)

    return out_p[:n, :Wfc.shape[1]]


# speedup vs baseline: 7.8046x; 1.1203x over previous
import dataclasses
import functools

import jax
import jax.numpy as jnp
from jax import lax
from jax.experimental import pallas as pl
from jax.experimental.pallas import tpu as pltpu
from jax.experimental.pallas import tpu_sc as plsc

NUM_CORES = 2
NUM_SUBCORES = 16
NUM_WORKERS = NUM_CORES * NUM_SUBCORES
CHUNK = 128
DEG_W = 128
ZROWS = 64


def _ceil_to(x, m):
    return (x + m - 1) // m * m


def _l1_body(x_ref, w_ref, b_ref, p_ref, r_ref):
    acc = jnp.dot(x_ref[...], w_ref[...], preferred_element_type=jnp.float32)
    d = p_ref.shape[2]
    p_ref[0] = acc[:, :d]
    p_ref[1] = acc[:, d:2 * d]
    r_ref[...] = acc[:, 2 * d:] + b_ref[...]


def _l2_body(agg_ref, deg_ref, r1_ref, w_ref, b_ref, p2_ref, r2_ref):
    deg = jnp.maximum(jnp.sum(deg_ref[...], axis=0), 1.0)[:, None]
    agg = jnp.concatenate([agg_ref[0], agg_ref[1]], axis=1)
    h = jnp.maximum(agg / deg + r1_ref[...], 0.0)
    acc = jnp.dot(h, w_ref[...], preferred_element_type=jnp.float32)
    d = p2_ref.shape[1]
    p2_ref[...] = acc[:, :d]
    r2_ref[...] = acc[:, d:] + b_ref[...]


def _head_body(agg2_ref, deg_ref, r2_ref, w_ref, b_ref, o_ref):
    deg = jnp.maximum(jnp.sum(deg_ref[...], axis=0), 1.0)[:, None]
    h2 = (agg2_ref[0] + agg2_ref[1]) / deg + r2_ref[...]
    o = jnp.dot(h2, w_ref[...], preferred_element_type=jnp.float32) + b_ref[...]
    o_ref[...] = jnp.where(o >= 0, o, 0.01 * o)


def _row_spec(bm, cols):
    return pl.BlockSpec((bm, cols), lambda i: (i, 0))


def _stk_spec(bm, cols):
    return pl.BlockSpec((2, bm, cols), lambda i: (0, i, 0))


def _deg_spec(bm):
    return pl.BlockSpec((NUM_WORKERS, bm), lambda i: (0, i))


def _full_spec(shape):
    nd = len(shape)
    return pl.BlockSpec(shape, lambda i, nd=nd: (0,) * nd)


def _edge_pass_pipelined(p_ref, src_v, dst_v, rows0, rows1, acc_sh,
                         gs0, gs1, ss0, ss1, n_chunks):
    def g(j, buf, sem):
        pltpu.async_copy(p_ref.at[src_v.at[j]], buf, sem)

    def gw(buf, sem):
        pltpu.make_async_copy(p_ref.at[src_v.at[0]], buf, sem).wait()

    def sct(j, buf, sem):
        pltpu.async_copy(buf, acc_sh.at[dst_v.at[j]], sem, add=True)

    def sw(buf, sem):
        pltpu.make_async_copy(buf, acc_sh.at[dst_v.at[0]], sem).wait()

    g(0, rows0, gs0)
    g(1, rows1, gs1)

    @pl.loop(0, n_chunks // 2 - 1)
    def _(t):
        j = 2 * t
        gw(rows0, gs0)
        sct(j, rows0, ss0)
        gw(rows1, gs1)
        sct(j + 1, rows1, ss1)
        sw(rows0, ss0)
        g(j + 2, rows0, gs0)
        sw(rows1, ss1)
        g(j + 3, rows1, gs1)

    gw(rows0, gs0)
    sct(n_chunks - 2, rows0, ss0)
    gw(rows1, gs1)
    sct(n_chunks - 1, rows1, ss1)
    sw(rows0, ss0)
    sw(rows1, ss1)


def _make_sc_deg(n_pad, n_chunks):
    mesh = plsc.VectorSubcoreMesh(core_axis_name="c", subcore_axis_name="s")
    f32 = jnp.float32

    cp = pltpu.CompilerParams()
    if "needs_layout_passes" in pltpu.CompilerParams.__dataclass_fields__:
        cp = dataclasses.replace(cp, needs_layout_passes=False)

    @functools.partial(
        pl.kernel,
        mesh=mesh,
        compiler_params=cp,
        out_type=[
            jax.ShapeDtypeStruct((NUM_WORKERS, n_pad), f32),
        ],
        scratch_types=[
            pltpu.VMEM((n_chunks, CHUNK), jnp.int32),
            pltpu.VMEM((n_pad,), f32),
        ],
    )
    def sc_kernel(dst_hbm, deg_hbm, dst_v, hist_v):
        c = lax.axis_index("c")
        s = lax.axis_index("s")
        wid = c * NUM_SUBCORES + s

        @pl.loop(0, n_pad // 16)
        def _(i):
            hist_v[pl.ds(i * 16, 16)] = jnp.zeros((16,), f32)

        pltpu.sync_copy(dst_hbm.at[wid], dst_v)

        @pl.loop(0, n_chunks)
        def _(j):
            for k in range(CHUNK // 16):
                idx = dst_v[j, pl.ds(k * 16, 16)]
                plsc.addupdate_scatter(hist_v, [idx], jnp.ones((16,), f32))

        pltpu.sync_copy(hist_v, deg_hbm.at[wid])

    return sc_kernel


def _make_sc_layer1(n_pad, n_chunks, d_half):
    rows_per_tile = n_pad // NUM_SUBCORES
    mesh = plsc.VectorSubcoreMesh(core_axis_name="c", subcore_axis_name="s")
    f32 = jnp.float32

    @functools.partial(
        pl.kernel,
        mesh=mesh,
        out_type=[
            jax.ShapeDtypeStruct((NUM_CORES, n_pad, d_half), f32),
        ],
        scratch_types=[
            pltpu.VMEM((n_chunks, CHUNK), jnp.int32),
            pltpu.VMEM((n_chunks, CHUNK), jnp.int32),
            pltpu.VMEM((CHUNK, d_half), f32),
            pltpu.VMEM((CHUNK, d_half), f32),
            pltpu.VMEM_SHARED((n_pad, d_half), f32),
            pltpu.SemaphoreType.DMA,
            pltpu.SemaphoreType.DMA,
            pltpu.SemaphoreType.DMA,
            pltpu.SemaphoreType.DMA,
        ],
    )
    def sc_kernel(p_hbm, src_hbm, dst_hbm, zeros_hbm, agg_hbm,
                  src_v, dst_v, rows0, rows1, acc_sh, gs0, gs1, ss0, ss1):
        c = lax.axis_index("c")
        s = lax.axis_index("s")

        rsl = pl.ds(s * rows_per_tile, rows_per_tile)
        pltpu.sync_copy(zeros_hbm, acc_sh.at[rsl])

        plsc.subcore_barrier()

        p_c = p_hbm.at[c]
        for rr in range(2):
            row = s * 2 + rr
            pltpu.sync_copy(src_hbm.at[row], src_v)
            pltpu.sync_copy(dst_hbm.at[row], dst_v)
            _edge_pass_pipelined(p_c, src_v, dst_v, rows0, rows1, acc_sh,
                                 gs0, gs1, ss0, ss1, n_chunks)

        plsc.subcore_barrier()

        pltpu.sync_copy(acc_sh.at[rsl], agg_hbm.at[c].at[rsl])

    return sc_kernel


def _make_sc_layer2(n_pad, n_chunks, d):
    rows_per_tile = n_pad // NUM_SUBCORES
    mesh = plsc.VectorSubcoreMesh(core_axis_name="c", subcore_axis_name="s")
    f32 = jnp.float32

    @functools.partial(
        pl.kernel,
        mesh=mesh,
        out_type=[
            jax.ShapeDtypeStruct((NUM_CORES, n_pad, d), f32),
        ],
        scratch_types=[
            pltpu.VMEM((n_chunks, CHUNK), jnp.int32),
            pltpu.VMEM((n_chunks, CHUNK), jnp.int32),
            pltpu.VMEM((CHUNK, d), f32),
            pltpu.VMEM((CHUNK, d), f32),
            pltpu.VMEM_SHARED((n_pad, d), f32),
            pltpu.SemaphoreType.DMA,
            pltpu.SemaphoreType.DMA,
            pltpu.SemaphoreType.DMA,
            pltpu.SemaphoreType.DMA,
        ],
    )
    def sc_kernel(p_hbm, src_hbm, dst_hbm, zeros_hbm, out_hbm,
                  src_v, dst_v, rows0, rows1, acc_sh, gs0, gs1, ss0, ss1):
        c = lax.axis_index("c")
        s = lax.axis_index("s")
        wid = c * NUM_SUBCORES + s

        rsl = pl.ds(s * rows_per_tile, rows_per_tile)
        pltpu.sync_copy(zeros_hbm, acc_sh.at[rsl])

        plsc.subcore_barrier()

        pltpu.sync_copy(src_hbm.at[wid], src_v)
        pltpu.sync_copy(dst_hbm.at[wid], dst_v)
        _edge_pass_pipelined(p_hbm, src_v, dst_v, rows0, rows1, acc_sh,
                             gs0, gs1, ss0, ss1, n_chunks)

        plsc.subcore_barrier()

        pltpu.sync_copy(acc_sh.at[rsl], out_hbm.at[c].at[rsl])

    return sc_kernel


def kernel(x, edge_index, W1_l, W1_r, b1, W2_l, W2_r, b2, Wfc, bfc):
    n, d_in = x.shape
    e = edge_index.shape[1]
    d_hid = W1_l.shape[1]
    d_out = W2_l.shape[1]
    f32 = jnp.float32

    n_pad = _ceil_to(n + 1, NUM_SUBCORES * ZROWS)
    e_pad = _ceil_to(e, NUM_WORKERS * CHUNK)
    n_chunks = e_pad // (NUM_WORKERS * CHUNK)
    bm = 512
    grid_m = n_pad // bm
    d_half = d_hid // 2

    x_p = jnp.pad(x.astype(f32), ((0, n_pad - n), (0, 0)))
    src = edge_index[0].astype(jnp.int32)
    dst = edge_index[1].astype(jnp.int32)
    pad_e = e_pad - e
    n_scratch = max(n_pad - n, 1)
    pad_i = jnp.arange(pad_e, dtype=jnp.int32)
    src = jnp.concatenate([src, pad_i % jnp.int32(n)])
    dst = jnp.concatenate([dst, (n_pad - n_scratch) + pad_i % jnp.int32(n_scratch)])
    src = src.reshape(NUM_WORKERS, n_chunks, CHUNK)
    dst = dst.reshape(NUM_WORKERS, n_chunks, CHUNK)

    w1 = jnp.concatenate([W1_l, W1_r], axis=1)
    b1_2d = b1.reshape(1, d_hid)
    w2 = jnp.concatenate([W2_l, W2_r], axis=1)
    b2_2d = b2.reshape(1, d_out)
    wfc_p = jnp.pad(Wfc, ((0, 0), (0, d_out - Wfc.shape[1])))
    bfc_p = jnp.pad(bfc, (0, d_out - bfc.shape[0])).reshape(1, d_out)

    rows_per_tile = n_pad // NUM_SUBCORES
    zeros_h = jnp.zeros((rows_per_tile, d_half), f32)

    scd = _make_sc_deg(n_pad, n_chunks)
    (deg,) = scd(dst)

    p1, r1 = pl.pallas_call(
        _l1_body,
        grid=(grid_m,),
        in_specs=[_row_spec(bm, d_in), _full_spec(w1.shape),
                  _full_spec(b1_2d.shape)],
        out_specs=[_stk_spec(bm, d_half), _row_spec(bm, d_hid)],
        out_shape=[jax.ShapeDtypeStruct((NUM_CORES, n_pad, d_half), f32),
                   jax.ShapeDtypeStruct((n_pad, d_hid), f32)],
    )(x_p, w1, b1_2d)

    sc1 = _make_sc_layer1(n_pad, n_chunks, d_half)
    (agg1,) = sc1(p1, src, dst, zeros_h)

    p2, r2 = pl.pallas_call(
        _l2_body,
        grid=(grid_m,),
        in_specs=[_stk_spec(bm, d_half), _deg_spec(bm),
                  _row_spec(bm, d_hid), _full_spec(w2.shape),
                  _full_spec(b2_2d.shape)],
        out_specs=[_row_spec(bm, d_out), _row_spec(bm, d_out)],
        out_shape=[jax.ShapeDtypeStruct((n_pad, d_out), f32),
                   jax.ShapeDtypeStruct((n_pad, d_out), f32)],
    )(agg1, deg, r1, w2, b2_2d)

    sc2 = _make_sc_layer2(n_pad, n_chunks, d_out)
    (agg2,) = sc2(p2, src, dst, zeros_h)

    out_p = pl.pallas_call(
        _head_body,
        grid=(grid_m,),
        in_specs=[_stk_spec(bm, d_out), _deg_spec(bm),
                  _row_spec(bm, d_out), _full_spec(wfc_p.shape),
                  _full_spec(bfc_p.shape)],
        out_specs=_row_spec(bm, d_out),
        out_shape=jax.ShapeDtypeStruct((n_pad, d_out), f32),
    )(agg2, deg, r2, wfc_p, bfc_p)

    return out_p[:n, :Wfc.shape[1]]


# speedup vs baseline: 7.8216x; 1.0022x over previous
import dataclasses
import functools

import jax
import jax.numpy as jnp
from jax import lax
from jax.experimental import pallas as pl
from jax.experimental.pallas import tpu as pltpu
from jax.experimental.pallas import tpu_sc as plsc

NUM_CORES = 2
NUM_SUBCORES = 16
NUM_WORKERS = NUM_CORES * NUM_SUBCORES
CHUNK = 128
ZROWS = 64


def _ceil_to(x, m):
    return (x + m - 1) // m * m


def _l1_body(x_ref, w_ref, b_ref, p_ref, r_ref):
    acc = jnp.dot(x_ref[...], w_ref[...], preferred_element_type=jnp.float32)
    d = p_ref.shape[2]
    p_ref[0] = acc[:, :d]
    p_ref[1] = acc[:, d:2 * d]
    r_ref[...] = acc[:, 2 * d:] + b_ref[...]


def _l2_body(agg_ref, deg_ref, r1_ref, w_ref, b_ref, p2_ref, r2_ref):
    deg = jnp.maximum(jnp.sum(deg_ref[...], axis=0), 1.0)[:, None]
    agg = jnp.concatenate([agg_ref[0], agg_ref[1]], axis=1)
    h = jnp.maximum(agg / deg + r1_ref[...], 0.0)
    acc = jnp.dot(h, w_ref[...], preferred_element_type=jnp.float32)
    d = p2_ref.shape[1]
    p2_ref[...] = acc[:, :d]
    r2_ref[...] = acc[:, d:] + b_ref[...]


def _head_body(agg2_ref, deg_ref, r2_ref, w_ref, b_ref, o_ref):
    deg = jnp.maximum(jnp.sum(deg_ref[...], axis=0), 1.0)[:, None]
    h2 = (agg2_ref[0] + agg2_ref[1]) / deg + r2_ref[...]
    o = jnp.dot(h2, w_ref[...], preferred_element_type=jnp.float32) + b_ref[...]
    o_ref[...] = jnp.where(o >= 0, o, 0.01 * o)


def _row_spec(bm, cols):
    return pl.BlockSpec((bm, cols), lambda i: (i, 0))


def _stk_spec(bm, cols):
    return pl.BlockSpec((2, bm, cols), lambda i: (0, i, 0))


def _deg_spec(bm):
    return pl.BlockSpec((NUM_WORKERS, bm), lambda i: (0, i))


def _full_spec(shape):
    nd = len(shape)
    return pl.BlockSpec(shape, lambda i, nd=nd: (0,) * nd)


def _edge_pass_pipelined(p_ref, src_v, dst_v, rows0, rows1, acc_sh,
                         gs0, gs1, ss0, ss1, n_chunks):
    def g(j, buf, sem):
        pltpu.async_copy(p_ref.at[src_v.at[j]], buf, sem)

    def gw(buf, sem):
        pltpu.make_async_copy(p_ref.at[src_v.at[0]], buf, sem).wait()

    def sct(j, buf, sem):
        pltpu.async_copy(buf, acc_sh.at[dst_v.at[j]], sem, add=True)

    def sw(buf, sem):
        pltpu.make_async_copy(buf, acc_sh.at[dst_v.at[0]], sem).wait()

    g(0, rows0, gs0)
    g(1, rows1, gs1)

    @pl.loop(0, n_chunks // 2 - 1)
    def _(t):
        j = 2 * t
        gw(rows0, gs0)
        sct(j, rows0, ss0)
        gw(rows1, gs1)
        sct(j + 1, rows1, ss1)
        sw(rows0, ss0)
        g(j + 2, rows0, gs0)
        sw(rows1, ss1)
        g(j + 3, rows1, gs1)

    gw(rows0, gs0)
    sct(n_chunks - 2, rows0, ss0)
    gw(rows1, gs1)
    sct(n_chunks - 1, rows1, ss1)
    sw(rows0, ss0)
    sw(rows1, ss1)


def _make_sc_deg(n_pad, n_chunks):
    mesh = plsc.VectorSubcoreMesh(core_axis_name="c", subcore_axis_name="s")
    f32 = jnp.float32

    cp = pltpu.CompilerParams()
    if "needs_layout_passes" in pltpu.CompilerParams.__dataclass_fields__:
        cp = dataclasses.replace(cp, needs_layout_passes=False)

    @functools.partial(
        pl.kernel,
        mesh=mesh,
        compiler_params=cp,
        out_type=[
            jax.ShapeDtypeStruct((NUM_WORKERS, n_pad), f32),
        ],
        scratch_types=[
            pltpu.VMEM((n_chunks, CHUNK), jnp.int32),
            pltpu.VMEM((n_pad,), f32),
        ],
    )
    def sc_kernel(dst_hbm, deg_hbm, dst_v, hist_v):
        c = lax.axis_index("c")
        s = lax.axis_index("s")
        wid = c * NUM_SUBCORES + s

        @pl.loop(0, n_pad // 16)
        def _(i):
            hist_v[pl.ds(i * 16, 16)] = jnp.zeros((16,), f32)

        pltpu.sync_copy(dst_hbm.at[wid], dst_v)

        @pl.loop(0, n_chunks)
        def _(j):
            for k in range(CHUNK // 16):
                idx = dst_v[j, pl.ds(k * 16, 16)]
                plsc.addupdate_scatter(hist_v, [idx], jnp.ones((16,), f32))

        pltpu.sync_copy(hist_v, deg_hbm.at[wid])

    return sc_kernel


def _make_sc_layer1(n_pad, n_chunks, d_half, dtype=jnp.float32):
    rows_per_tile = n_pad // NUM_SUBCORES
    mesh = plsc.VectorSubcoreMesh(core_axis_name="c", subcore_axis_name="s")
    f32 = jnp.float32

    @functools.partial(
        pl.kernel,
        mesh=mesh,
        out_type=[
            jax.ShapeDtypeStruct((NUM_CORES, n_pad, d_half), dtype),
        ],
        scratch_types=[
            pltpu.VMEM((n_chunks, CHUNK), jnp.int32),
            pltpu.VMEM((n_chunks, CHUNK), jnp.int32),
            pltpu.VMEM((CHUNK, d_half), dtype),
            pltpu.VMEM((CHUNK, d_half), dtype),
            pltpu.VMEM_SHARED((n_pad, d_half), dtype),
            pltpu.SemaphoreType.DMA,
            pltpu.SemaphoreType.DMA,
            pltpu.SemaphoreType.DMA,
            pltpu.SemaphoreType.DMA,
        ],
    )
    def sc_kernel(p_hbm, src_hbm, dst_hbm, zeros_hbm, agg_hbm,
                  src_v, dst_v, rows0, rows1, acc_sh, gs0, gs1, ss0, ss1):
        c = lax.axis_index("c")
        s = lax.axis_index("s")

        rsl = pl.ds(s * rows_per_tile, rows_per_tile)
        pltpu.sync_copy(zeros_hbm, acc_sh.at[rsl])

        plsc.subcore_barrier()

        p_c = p_hbm.at[c]
        for rr in range(2):
            row = s * 2 + rr
            pltpu.sync_copy(src_hbm.at[row], src_v)
            pltpu.sync_copy(dst_hbm.at[row], dst_v)
            _edge_pass_pipelined(p_c, src_v, dst_v, rows0, rows1, acc_sh,
                                 gs0, gs1, ss0, ss1, n_chunks)

        plsc.subcore_barrier()

        pltpu.sync_copy(acc_sh.at[rsl], agg_hbm.at[c].at[rsl])

    return sc_kernel


def _make_sc_layer2(n_pad, n_chunks, d, dtype=jnp.float32):
    rows_per_tile = n_pad // NUM_SUBCORES
    mesh = plsc.VectorSubcoreMesh(core_axis_name="c", subcore_axis_name="s")
    f32 = jnp.float32

    @functools.partial(
        pl.kernel,
        mesh=mesh,
        out_type=[
            jax.ShapeDtypeStruct((NUM_CORES, n_pad, d), dtype),
        ],
        scratch_types=[
            pltpu.VMEM((n_chunks, CHUNK), jnp.int32),
            pltpu.VMEM((n_chunks, CHUNK), jnp.int32),
            pltpu.VMEM((CHUNK, d), dtype),
            pltpu.VMEM((CHUNK, d), dtype),
            pltpu.VMEM_SHARED((n_pad, d), dtype),
            pltpu.SemaphoreType.DMA,
            pltpu.SemaphoreType.DMA,
            pltpu.SemaphoreType.DMA,
            pltpu.SemaphoreType.DMA,
        ],
    )
    def sc_kernel(p_hbm, src_hbm, dst_hbm, zeros_hbm, out_hbm,
                  src_v, dst_v, rows0, rows1, acc_sh, gs0, gs1, ss0, ss1):
        c = lax.axis_index("c")
        s = lax.axis_index("s")
        wid = c * NUM_SUBCORES + s

        rsl = pl.ds(s * rows_per_tile, rows_per_tile)
        pltpu.sync_copy(zeros_hbm, acc_sh.at[rsl])

        plsc.subcore_barrier()

        pltpu.sync_copy(src_hbm.at[wid], src_v)
        pltpu.sync_copy(dst_hbm.at[wid], dst_v)
        _edge_pass_pipelined(p_hbm, src_v, dst_v, rows0, rows1, acc_sh,
                             gs0, gs1, ss0, ss1, n_chunks)

        plsc.subcore_barrier()

        pltpu.sync_copy(acc_sh.at[rsl], out_hbm.at[c].at[rsl])

    return sc_kernel


def kernel(x, edge_index, W1_l, W1_r, b1, W2_l, W2_r, b2, Wfc, bfc):
    n, d_in = x.shape
    e = edge_index.shape[1]
    d_hid = W1_l.shape[1]
    d_out = W2_l.shape[1]
    f32 = jnp.float32

    n_pad = _ceil_to(n + 1, NUM_SUBCORES * ZROWS)
    e_pad = _ceil_to(e, NUM_WORKERS * CHUNK)
    n_chunks = e_pad // (NUM_WORKERS * CHUNK)
    bm = 512
    grid_m = n_pad // bm
    d_half = d_hid // 2

    x_p = jnp.pad(x.astype(f32), ((0, n_pad - n), (0, 0)))
    src = edge_index[0].astype(jnp.int32)
    dst = edge_index[1].astype(jnp.int32)
    pad_e = e_pad - e
    n_scratch = max(n_pad - n, 1)
    pad_i = jnp.arange(pad_e, dtype=jnp.int32)
    src = jnp.concatenate([src, pad_i % jnp.int32(n)])
    dst = jnp.concatenate([dst, (n_pad - n_scratch) + pad_i % jnp.int32(n_scratch)])
    src = src.reshape(NUM_WORKERS, n_chunks, CHUNK)
    dst = dst.reshape(NUM_WORKERS, n_chunks, CHUNK)

    w1 = jnp.concatenate([W1_l, W1_r], axis=1)
    b1_2d = b1.reshape(1, d_hid)
    w2 = jnp.concatenate([W2_l, W2_r], axis=1)
    b2_2d = b2.reshape(1, d_out)
    wfc_p = jnp.pad(Wfc, ((0, 0), (0, d_out - Wfc.shape[1])))
    bfc_p = jnp.pad(bfc, (0, d_out - bfc.shape[0])).reshape(1, d_out)

    rows_per_tile = n_pad // NUM_SUBCORES
    zeros_h = jnp.zeros((rows_per_tile, d_half), f32)

    scd = _make_sc_deg(n_pad, n_chunks)
    (deg,) = scd(dst)

    p1, r1 = pl.pallas_call(
        _l1_body,
        grid=(grid_m,),
        in_specs=[_row_spec(bm, d_in), _full_spec(w1.shape),
                  _full_spec(b1_2d.shape)],
        out_specs=[_stk_spec(bm, d_half), _row_spec(bm, d_hid)],
        out_shape=[jax.ShapeDtypeStruct((NUM_CORES, n_pad, d_half), f32),
                   jax.ShapeDtypeStruct((n_pad, d_hid), f32)],
    )(x_p, w1, b1_2d)

    sc1 = _make_sc_layer1(n_pad, n_chunks, d_half)
    (agg1,) = sc1(p1, src, dst, zeros_h)

    p2, r2 = pl.pallas_call(
        _l2_body,
        grid=(grid_m,),
        in_specs=[_stk_spec(bm, d_half), _deg_spec(bm),
                  _row_spec(bm, d_hid), _full_spec(w2.shape),
                  _full_spec(b2_2d.shape)],
        out_specs=[_row_spec(bm, d_out), _row_spec(bm, d_out)],
        out_shape=[jax.ShapeDtypeStruct((n_pad, d_out), f32),
                   jax.ShapeDtypeStruct((n_pad, d_out), f32)],
    )(agg1, deg, r1, w2, b2_2d)

    sc2 = _make_sc_layer2(n_pad, n_chunks, d_out)
    (agg2,) = sc2(p2, src, dst, zeros_h)

    out_p = pl.pallas_call(
        _head_body,
        grid=(grid_m,),
        in_specs=[_stk_spec(bm, d_out), _deg_spec(bm),
                  _row_spec(bm, d_out), _full_spec(wfc_p.shape),
                  _full_spec(bfc_p.shape)],
        out_specs=_row_spec(bm, d_out),
        out_shape=jax.ShapeDtypeStruct((n_pad, d_out), f32),
    )(agg2, deg, r2, wfc_p, bfc_p)

    return out_p[:n, :Wfc.shape[1]]
